# Initial kernel scaffold; baseline (speedup 1.0000x reference)
import jax
import jax.numpy as jnp
from jax.experimental import pallas as pl


def kernel(features, labels, prototypes):
    raise NotImplementedError("write your pallas kernel here")


# speedup vs baseline: 216.8765x; 216.8765x over previous
import jax
import jax.numpy as jnp
from jax import lax
from jax.experimental import pallas as pl
from jax.experimental.pallas import tpu as pltpu
from jax.experimental.pallas import tpu_sc as plsc

NUM_CLASSES = 100000
FEAT_DIM = 128
BATCH = 16384

NC = 2
NS = 16
NW = NC * NS
LANES = 16
NVEC = BATCH // LANES


def _rsqrt(x):
    i = plsc.bitcast(x, jnp.int32)
    i = 0x5F3759DF - lax.shift_right_logical(i, 1)
    y = plsc.bitcast(i, jnp.float32)
    for _ in range(3):
        y = y * (1.5 - 0.5 * x * y * y)
    return y


def _body(feat_hbm, lbl_hbm, proto_hbm,
          lbl_v, own_idx_v, own_lbl_v, frow_v, nrow_v):
    wid = lax.axis_index("s") * NC + lax.axis_index("c")
    wid = wid.astype(jnp.int32)

    pltpu.sync_copy(lbl_hbm, lbl_v)

    def scan_step(i, n):
        lbl = lbl_v[pl.ds(i * LANES, LANES)]
        mask = (lbl & (NW - 1)) == wid
        cnt = jnp.where(mask, jnp.ones((LANES,), jnp.int32),
                        jnp.zeros((LANES,), jnp.int32))
        pos = n + plsc.cumsum(cnt) - 1
        idx = lax.iota(jnp.int32, LANES) + i * LANES
        plsc.store_scatter(own_idx_v, [pos], idx, mask=mask)
        plsc.store_scatter(own_lbl_v, [pos], lbl, mask=mask)
        return n + lax.reduce_sum(cnt, axes=(0,))

    n_own = lax.fori_loop(0, NVEC, scan_step, jnp.int32(0))

    def step(j, carry):
        l = own_lbl_v[pl.ds(j, LANES)][0]
        s = own_idx_v[pl.ds(j, LANES)][0]
        pltpu.sync_copy(feat_hbm.at[s], frow_v)
        pltpu.sync_copy(proto_hbm.at[l], nrow_v)
        acc = jnp.zeros((LANES,), jnp.float32)
        for kk in range(FEAT_DIM // LANES):
            sl = pl.ds(kk * LANES, LANES)
            nv = (nrow_v[sl] + frow_v[sl]) * 0.5
            nrow_v[sl] = nv
            acc = acc + nv * nv
        s2 = lax.reduce_sum(acc, axes=(0,))
        s2v = jnp.maximum(jnp.full((LANES,), s2), 1e-30)
        nrm = jnp.maximum(s2v * _rsqrt(s2v), 1e-12)
        inv = 1.0 / nrm
        for kk in range(FEAT_DIM // LANES):
            sl = pl.ds(kk * LANES, LANES)
            nrow_v[sl] = nrow_v[sl] * inv
        pltpu.sync_copy(nrow_v, proto_hbm.at[l])
        return carry

    lax.fori_loop(0, n_own, step, jnp.int32(0))


_sc_update = pl.kernel(
    _body,
    out_type=(),
    mesh=plsc.VectorSubcoreMesh(core_axis_name="c", subcore_axis_name="s"),
    compiler_params=pltpu.CompilerParams(needs_layout_passes=False),
    scratch_types=[
        pltpu.VMEM((BATCH,), jnp.int32),
        pltpu.VMEM((BATCH + LANES,), jnp.int32),
        pltpu.VMEM((BATCH + LANES,), jnp.int32),
        pltpu.VMEM((FEAT_DIM,), jnp.float32),
        pltpu.VMEM((FEAT_DIM,), jnp.float32),
    ],
)


def kernel(features, labels, prototypes):
    labels = labels.astype(jnp.int32)
    proto_ref = jax.new_ref(prototypes)
    _sc_update(features, labels, proto_ref)
    return proto_ref[...]


# speedup vs baseline: 430.1136x; 1.9832x over previous
import jax
import jax.numpy as jnp
from jax import lax
from jax.experimental import pallas as pl
from jax.experimental.pallas import tpu as pltpu
from jax.experimental.pallas import tpu_sc as plsc

NUM_CLASSES = 100000
FEAT_DIM = 128
BATCH = 16384

NC = 2
NS = 16
NW = NC * NS
LANES = 16
NVEC = BATCH // LANES
NCHUNK = FEAT_DIM // LANES


def _rsqrt(x):
    i = plsc.bitcast(x, jnp.int32)
    i = 0x5F3759DF - lax.shift_right_logical(i, 1)
    y = plsc.bitcast(i, jnp.float32)
    for _ in range(3):
        y = y * (1.5 - 0.5 * x * y * y)
    return y


def _body(feat_hbm, lbl_hbm, proto_hbm,
          lbl_v, own_idx_v, own_lbl_v, tbl_v,
          prows_v, frows_v, nrows_v, frow_v, nrow_v,
          perm_v, dupj_v, sem1, sem2):
    wid = lax.axis_index("s") * NC + lax.axis_index("c")
    wid = wid.astype(jnp.int32)

    pltpu.sync_copy(lbl_hbm, lbl_v)

    def scan_step(i, n):
        lbl = lbl_v[pl.ds(i * LANES, LANES)]
        mask = (lbl & (NW - 1)) == wid
        cnt = jnp.where(mask, jnp.ones((LANES,), jnp.int32),
                        jnp.zeros((LANES,), jnp.int32))
        pos = n + plsc.cumsum(cnt) - 1
        idx = lax.iota(jnp.int32, LANES) + i * LANES
        plsc.store_scatter(own_idx_v, [pos], idx, mask=mask)
        plsc.store_scatter(own_lbl_v, [pos], lbl, mask=mask)
        return n + lax.reduce_sum(cnt, axes=(0,))

    n_own = lax.fori_loop(0, NVEC, scan_step, jnp.int32(0))

    def serial_one(j):
        l = own_lbl_v[pl.ds(j, LANES)][0]
        s = own_idx_v[pl.ds(j, LANES)][0]
        pltpu.sync_copy(feat_hbm.at[s], frow_v)
        pltpu.sync_copy(proto_hbm.at[l], nrow_v)
        acc = jnp.zeros((LANES,), jnp.float32)
        for kk in range(NCHUNK):
            sl = pl.ds(kk * LANES, LANES)
            nv = (nrow_v[sl] + frow_v[sl]) * 0.5
            nrow_v[sl] = nv
            acc = acc + nv * nv
        s2 = lax.reduce_sum(acc, axes=(0,))
        s2v = jnp.maximum(jnp.full((LANES,), s2), 1e-30)
        inv = 1.0 / jnp.maximum(s2v * _rsqrt(s2v), 1e-12)
        for kk in range(NCHUNK):
            sl = pl.ds(kk * LANES, LANES)
            nrow_v[sl] = nrow_v[sl] * inv
        pltpu.sync_copy(nrow_v, proto_hbm.at[l])

    rows16 = lax.iota(jnp.int32, LANES)

    def grp(g, carry):
        base = g * LANES
        lblv = own_lbl_v[pl.ds(base, LANES)]
        idxv = own_idx_v[pl.ds(base, LANES)]
        lid = lax.shift_right_logical(lblv, 5)
        plsc.store_scatter(tbl_v, [lax.rev(lid, (0,))],
                           lax.rev(rows16, (0,)))
        first = plsc.load_gather(tbl_v, [lid])
        dmask = first != rows16
        ndup = plsc.all_reduce_population_count(dmask)[0]

        plsc.store_compressed(dupj_v.at[pl.ds(0, LANES)], base + rows16,
                              mask=dmask)

        cp1 = pltpu.async_copy(proto_hbm.at[lblv], prows_v, sem1)
        cp2 = pltpu.async_copy(feat_hbm.at[idxv], frows_v, sem2)
        cp1.wait()
        cp2.wait()

        def fast():
            def col(c, acc):
                cols = jnp.full((LANES,), c, jnp.int32)
                pv = plsc.load_gather(prows_v, [rows16, cols])
                fv = plsc.load_gather(frows_v, [rows16, cols])
                nv = (pv + fv) * 0.5
                plsc.store_scatter(nrows_v, [rows16, cols], nv)
                return acc + nv * nv

            acc = lax.fori_loop(0, FEAT_DIM, col,
                                jnp.zeros((LANES,), jnp.float32), unroll=8)
            x = jnp.maximum(acc, 1e-30)
            inv = 1.0 / jnp.maximum(x * _rsqrt(x), 1e-12)

            def col2(c, carry):
                cols = jnp.full((LANES,), c, jnp.int32)
                nv = plsc.load_gather(nrows_v, [rows16, cols])
                plsc.store_scatter(nrows_v, [rows16, cols], nv * inv)
                return carry

            lax.fori_loop(0, FEAT_DIM, col2, jnp.int32(0), unroll=8)
            pltpu.async_copy(nrows_v, proto_hbm.at[lblv], sem1).wait()

        def fastdup():
            def col(c, acc):
                cols = jnp.full((LANES,), c, jnp.int32)
                pv = plsc.load_gather(prows_v, [rows16, cols])
                fv = plsc.load_gather(frows_v, [rows16, cols])
                nv = (pv + fv) * 0.5
                perm_v[pl.ds(0, LANES)] = nv
                nvp = plsc.load_gather(perm_v, [first])
                plsc.store_scatter(nrows_v, [rows16, cols], nvp)
                return acc + nvp * nvp

            acc = lax.fori_loop(0, FEAT_DIM, col,
                                jnp.zeros((LANES,), jnp.float32), unroll=8)
            x = jnp.maximum(acc, 1e-30)
            inv = 1.0 / jnp.maximum(x * _rsqrt(x), 1e-12)

            def col2(c, carry):
                cols = jnp.full((LANES,), c, jnp.int32)
                nv = plsc.load_gather(nrows_v, [rows16, cols])
                plsc.store_scatter(nrows_v, [rows16, cols], nv * inv)
                return carry

            lax.fori_loop(0, FEAT_DIM, col2, jnp.int32(0), unroll=8)
            pltpu.async_copy(nrows_v, proto_hbm.at[lblv], sem1).wait()

            def fstep(k, carry):
                serial_one(dupj_v[pl.ds(k, LANES)][0])
                return carry

            lax.fori_loop(0, ndup, fstep, jnp.int32(0))

        lax.cond(ndup == 0, fast, fastdup)
        return carry

    n_full = n_own // LANES
    lax.fori_loop(0, n_full, grp, jnp.int32(0))

    def tstep(j, carry):
        serial_one(j)
        return carry

    lax.fori_loop(n_full * LANES, n_own, tstep, jnp.int32(0))


_sc_update = pl.kernel(
    _body,
    out_type=(),
    mesh=plsc.VectorSubcoreMesh(core_axis_name="c", subcore_axis_name="s"),
    compiler_params=pltpu.CompilerParams(needs_layout_passes=False),
    scratch_types=[
        pltpu.VMEM((BATCH,), jnp.int32),
        pltpu.VMEM((BATCH + LANES,), jnp.int32),
        pltpu.VMEM((BATCH + LANES,), jnp.int32),
        pltpu.VMEM((NUM_CLASSES // NW + LANES,), jnp.int32),
        pltpu.VMEM((LANES, FEAT_DIM), jnp.float32),
        pltpu.VMEM((LANES, FEAT_DIM), jnp.float32),
        pltpu.VMEM((LANES, FEAT_DIM), jnp.float32),
        pltpu.VMEM((FEAT_DIM,), jnp.float32),
        pltpu.VMEM((FEAT_DIM,), jnp.float32),
        pltpu.VMEM((LANES,), jnp.float32),
        pltpu.VMEM((2 * LANES,), jnp.int32),
        pltpu.SemaphoreType.DMA,
        pltpu.SemaphoreType.DMA,
    ],
)


def kernel(features, labels, prototypes):
    labels = labels.astype(jnp.int32)
    proto_ref = jax.new_ref(prototypes)
    _sc_update(features, labels, proto_ref)
    return proto_ref[...]


# speedup vs baseline: 439.0669x; 1.0208x over previous
import jax
import jax.numpy as jnp
from jax import lax
from jax.experimental import pallas as pl
from jax.experimental.pallas import tpu as pltpu
from jax.experimental.pallas import tpu_sc as plsc

NUM_CLASSES = 100000
FEAT_DIM = 128
BATCH = 16384

NC = 2
NS = 16
NW = NC * NS
LANES = 16
NVEC = BATCH // LANES
NCHUNK = FEAT_DIM // LANES
GW = 64
NSUB = GW // LANES


def _rsqrt(x):
    i = plsc.bitcast(x, jnp.int32)
    i = 0x5F3759DF - lax.shift_right_logical(i, 1)
    y = plsc.bitcast(i, jnp.float32)
    for _ in range(3):
        y = y * (1.5 - 0.5 * x * y * y)
    return y


def _body(feat_hbm, lbl_hbm, proto_hbm,
          lbl_v, own_idx_v, own_lbl_v, tbl_v,
          prows_v, frows_v, nrows_v, frow_v, nrow_v,
          perm_v, dupj_v, sem1, sem2):
    wid = lax.axis_index("s") * NC + lax.axis_index("c")
    wid = wid.astype(jnp.int32)

    pltpu.sync_copy(lbl_hbm, lbl_v)

    def scan_step(i, n):
        lbl = lbl_v[pl.ds(i * LANES, LANES)]
        mask = (lbl & (NW - 1)) == wid
        plsc.store_compressed(own_lbl_v.at[pl.ds(n, LANES)], lbl, mask=mask)
        idx = lax.iota(jnp.int32, LANES) + i * LANES
        plsc.store_compressed(own_idx_v.at[pl.ds(n, LANES)], idx, mask=mask)
        return n + plsc.all_reduce_population_count(mask)[0]

    n_own = lax.fori_loop(0, NVEC, scan_step, jnp.int32(0), unroll=4)

    rows16 = lax.iota(jnp.int32, LANES)

    @pl.when(n_own > 0)
    def _pad():
        last_l = own_lbl_v[pl.ds(n_own - 1, LANES)][0]
        last_i = own_idx_v[pl.ds(n_own - 1, LANES)][0]
        for t in range(NSUB):
            own_lbl_v[pl.ds(n_own + t * LANES, LANES)] = jnp.full(
                (LANES,), last_l)
            own_idx_v[pl.ds(n_own + t * LANES, LANES)] = jnp.full(
                (LANES,), last_i)

    def serial_one(j):
        l = own_lbl_v[pl.ds(j, LANES)][0]
        s = own_idx_v[pl.ds(j, LANES)][0]
        pltpu.sync_copy(feat_hbm.at[s], frow_v)
        pltpu.sync_copy(proto_hbm.at[l], nrow_v)
        acc = jnp.zeros((LANES,), jnp.float32)
        for kk in range(NCHUNK):
            sl = pl.ds(kk * LANES, LANES)
            nv = (nrow_v[sl] + frow_v[sl]) * 0.5
            nrow_v[sl] = nv
            acc = acc + nv * nv
        s2 = lax.reduce_sum(acc, axes=(0,))
        s2v = jnp.maximum(jnp.full((LANES,), s2), 1e-30)
        inv = 1.0 / jnp.maximum(s2v * _rsqrt(s2v), 1e-12)
        for kk in range(NCHUNK):
            sl = pl.ds(kk * LANES, LANES)
            nrow_v[sl] = nrow_v[sl] * inv
        pltpu.sync_copy(nrow_v, proto_hbm.at[l])

    def grp(g, carry):
        base = g * GW
        lbls = [own_lbl_v[pl.ds(base + t * LANES, LANES)]
                for t in range(NSUB)]
        idxs = [own_idx_v[pl.ds(base + t * LANES, LANES)]
                for t in range(NSUB)]

        for t in reversed(range(NSUB)):
            lid_t = lax.shift_right_logical(lbls[t], 5)
            plsc.store_scatter(tbl_v, [lax.rev(lid_t, (0,))],
                               lax.rev(rows16, (0,)) + t * LANES)
        firsts = [plsc.load_gather(tbl_v, [lax.shift_right_logical(lbls[t], 5)])
                  for t in range(NSUB)]

        n_d = jnp.int32(0)
        for t in range(NSUB):
            gl = rows16 + t * LANES
            dmask = (firsts[t] != gl) & (base + gl < n_own)
            plsc.store_compressed(dupj_v.at[pl.ds(n_d, LANES)],
                                  base + gl, mask=dmask)
            n_d = n_d + plsc.all_reduce_population_count(dmask)[0]

        cp1 = pltpu.async_copy(
            proto_hbm.at[own_lbl_v.at[pl.ds(base, GW)]], prows_v, sem1)
        cp2 = pltpu.async_copy(
            feat_hbm.at[own_idx_v.at[pl.ds(base, GW)]], frows_v, sem2)
        cp1.wait()
        cp2.wait()

        accs = [jnp.zeros((LANES,), jnp.float32) for _ in range(NSUB)]

        def col(c, accs):
            cols = jnp.full((LANES,), c, jnp.int32)
            nvs = []
            for t in range(NSUB):
                rt = rows16 + t * LANES
                pv = plsc.load_gather(prows_v, [rt, cols])
                fv = plsc.load_gather(frows_v, [rt, cols])
                nvs.append((pv + fv) * 0.5)
            for t in range(NSUB):
                perm_v[pl.ds(t * LANES, LANES)] = nvs[t]
            out = []
            for t in range(NSUB):
                nvp = plsc.load_gather(perm_v, [firsts[t]])
                plsc.store_scatter(nrows_v, [rows16 + t * LANES, cols], nvp)
                out.append(accs[t] + nvp * nvp)
            return out

        accs = lax.fori_loop(0, FEAT_DIM, col, accs, unroll=4)
        invs = []
        for t in range(NSUB):
            x = jnp.maximum(accs[t], 1e-30)
            invs.append(1.0 / jnp.maximum(x * _rsqrt(x), 1e-12))

        def col2(c, carry):
            cols = jnp.full((LANES,), c, jnp.int32)
            for t in range(NSUB):
                rt = rows16 + t * LANES
                nv = plsc.load_gather(nrows_v, [rt, cols])
                plsc.store_scatter(nrows_v, [rt, cols], nv * invs[t])
            return carry

        lax.fori_loop(0, FEAT_DIM, col2, jnp.int32(0), unroll=4)

        cps = [pltpu.async_copy(nrows_v.at[pl.ds(t * LANES, LANES)],
                                proto_hbm.at[lbls[t]], sem1)
               for t in range(NSUB)]
        for cp in cps:
            cp.wait()

        def fstep(k, carry):
            serial_one(dupj_v[pl.ds(k, LANES)][0])
            return carry

        lax.fori_loop(0, n_d, fstep, jnp.int32(0))
        return carry

    n_grp = (n_own + GW - 1) // GW
    lax.fori_loop(0, n_grp, grp, jnp.int32(0))


_sc_update = pl.kernel(
    _body,
    out_type=(),
    mesh=plsc.VectorSubcoreMesh(core_axis_name="c", subcore_axis_name="s"),
    compiler_params=pltpu.CompilerParams(needs_layout_passes=False),
    scratch_types=[
        pltpu.VMEM((BATCH,), jnp.int32),
        pltpu.VMEM((BATCH + 2 * GW,), jnp.int32),
        pltpu.VMEM((BATCH + 2 * GW,), jnp.int32),
        pltpu.VMEM((NUM_CLASSES // NW + LANES,), jnp.int32),
        pltpu.VMEM((GW, FEAT_DIM), jnp.float32),
        pltpu.VMEM((GW, FEAT_DIM), jnp.float32),
        pltpu.VMEM((GW, FEAT_DIM), jnp.float32),
        pltpu.VMEM((FEAT_DIM,), jnp.float32),
        pltpu.VMEM((FEAT_DIM,), jnp.float32),
        pltpu.VMEM((GW,), jnp.float32),
        pltpu.VMEM((GW + LANES,), jnp.int32),
        pltpu.SemaphoreType.DMA,
        pltpu.SemaphoreType.DMA,
    ],
)


def kernel(features, labels, prototypes):
    labels = labels.astype(jnp.int32)
    proto_ref = jax.new_ref(prototypes)
    _sc_update(features, labels, proto_ref)
    return proto_ref[...]
